# baseline (speedup 1.0000x reference)
import jax
import jax.numpy as jnp
from jax import lax
from jax.experimental import pallas as pl
from jax.experimental.pallas import tpu as pltpu
from jax.experimental.pallas import tpu_sc as plsc

N = 100000
E = 3200000
EDGE_DIM = 16
K = 5

NC, NS, L = 2, 16, 16
NW = NC * NS
NPAD = 100352
E_PER_TILE = E // NW
CHUNK = 10000
N_CHUNKS = E_PER_TILE // CHUNK
NSLICE = NPAD // NS
NROWS = NPAD // 128

_mesh = plsc.VectorSubcoreMesh(core_axis_name="c", subcore_axis_name="s")


def _fill_zero(zv):
    zero = jnp.zeros((L,), jnp.float32)

    def zloop(i, carry):
        zv[pl.ds(i * L, L)] = zero
        return carry

    lax.fori_loop(0, NSLICE // L, zloop, 0)


def _scatter_chunks(wid, row_hbm, col_hbm, w_hbm, row_v, col_v, w_v,
                    gath_v, y_sh, acc, sem, gather):
    base = wid * E_PER_TILE

    def chunk(j, carry):
        e0 = base + j * CHUNK
        if gather:
            c1 = pltpu.async_copy(row_hbm.at[pl.ds(e0, CHUNK)], row_v, sem)
        c3 = pltpu.async_copy(w_hbm.at[pl.ds(e0, CHUNK)], w_v, sem)
        c2 = pltpu.async_copy(col_hbm.at[pl.ds(e0, CHUNK)], col_v, sem)
        if gather:
            c1.wait()
            pltpu.sync_copy(y_sh.at[row_v], gath_v)
            c3.wait()

            def mul_group(g, c_):
                s = pl.ds(g * L, L)
                gath_v[s] = gath_v[s] * w_v[s]
                return c_

            lax.fori_loop(0, CHUNK // L, mul_group, 0)
            src = gath_v
        else:
            c3.wait()
            src = w_v
        c2.wait()
        pltpu.sync_copy(src, acc.at[col_v], add=True)
        return carry

    lax.fori_loop(0, N_CHUNKS, chunk, 0)


def _writeback(cid, sid, acc, out0, out1):
    sl = pl.ds(sid * NSLICE, NSLICE)

    @pl.when(cid == 0)
    def _():
        pltpu.sync_copy(acc.at[sl], out0.at[sl])

    @pl.when(cid == 1)
    def _():
        pltpu.sync_copy(acc.at[sl], out1.at[sl])


def _deg_body(col_hbm, w_hbm, out0, out1, col_v, w_v, zv, acc, sem):
    cid = lax.axis_index("c")
    sid = lax.axis_index("s")
    wid = cid * NS + sid
    _fill_zero(zv)
    pltpu.sync_copy(zv, acc.at[pl.ds(sid * NSLICE, NSLICE)])
    plsc.subcore_barrier()
    _scatter_chunks(wid, None, col_hbm, w_hbm, None, col_v, w_v,
                    None, None, acc, sem, gather=False)
    plsc.subcore_barrier()
    _writeback(cid, sid, acc, out0, out1)


_deg_call = pl.kernel(
    _deg_body,
    out_type=[jax.ShapeDtypeStruct((NPAD,), jnp.float32)] * 2,
    mesh=_mesh,
    scratch_types=[
        pltpu.VMEM((CHUNK,), jnp.int32),
        pltpu.VMEM((CHUNK,), jnp.float32),
        pltpu.VMEM((NSLICE,), jnp.float32),
        pltpu.VMEM_SHARED((NPAD,), jnp.float32),
        pltpu.SemaphoreType.DMA,
    ],
)


def _make_prop(first):
    if first:
        def body(y_hbm, row_hbm, col_hbm, w_hbm, out0, out1,
                 row_v, col_v, w_v, gath_v, zv, y_sh, acc, sem):
            cid = lax.axis_index("c")
            sid = lax.axis_index("s")
            wid = cid * NS + sid
            sl = pl.ds(sid * NSLICE, NSLICE)
            pltpu.sync_copy(y_hbm.at[sl], zv)
            pltpu.sync_copy(zv, y_sh.at[sl])
            _fill_zero(zv)
            pltpu.sync_copy(zv, acc.at[sl])
            plsc.subcore_barrier()
            _scatter_chunks(wid, row_hbm, col_hbm, w_hbm, row_v, col_v,
                            w_v, gath_v, y_sh, acc, sem, gather=True)
            plsc.subcore_barrier()
            _writeback(cid, sid, acc, out0, out1)

        extra_in = 0
    else:
        def body(alpha_hbm, a0_hbm, a1_hbm, xp_hbm, dis_hbm, dinv_hbm,
                 h_hbm, row_hbm, col_hbm, w_hbm, out0, out1, xn_hbm,
                 row_v, col_v, w_v, gath_v, zv,
                 a0_v, a1_v, xp_v, dis_v, dinv_v, h_v, av,
                 y_sh, acc, sem):
            cid = lax.axis_index("c")
            sid = lax.axis_index("s")
            wid = cid * NS + sid
            sl = pl.ds(sid * NSLICE, NSLICE)
            pltpu.sync_copy(alpha_hbm, av)
            d1 = pltpu.async_copy(a0_hbm.at[sl], a0_v, sem)
            d2 = pltpu.async_copy(a1_hbm.at[sl], a1_v, sem)
            d3 = pltpu.async_copy(xp_hbm.at[sl], xp_v, sem)
            d4 = pltpu.async_copy(dis_hbm.at[sl], dis_v, sem)
            d5 = pltpu.async_copy(dinv_hbm.at[sl], dinv_v, sem)
            d6 = pltpu.async_copy(h_hbm.at[sl], h_v, sem)
            alpha = av[pl.ds(0, L)]
            one_m_alpha = 1.0 - alpha
            d1.wait(); d2.wait(); d3.wait(); d4.wait(); d5.wait(); d6.wait()

            def up_group(g, carry):
                s = pl.ds(g * L, L)
                agg = dis_v[s] * (a0_v[s] + a1_v[s]) + xp_v[s] * dinv_v[s]
                xn = agg * one_m_alpha + alpha * h_v[s]
                xp_v[s] = xn
                a0_v[s] = dis_v[s] * xn
                return carry

            lax.fori_loop(0, NSLICE // L, up_group, 0)
            pltpu.sync_copy(a0_v, y_sh.at[sl])

            @pl.when(cid == 0)
            def _():
                pltpu.sync_copy(xp_v, xn_hbm.at[sl])

            _fill_zero(zv)
            pltpu.sync_copy(zv, acc.at[sl])
            plsc.subcore_barrier()
            _scatter_chunks(wid, row_hbm, col_hbm, w_hbm, row_v, col_v,
                            w_v, gath_v, y_sh, acc, sem, gather=True)
            plsc.subcore_barrier()
            _writeback(cid, sid, acc, out0, out1)

        extra_in = 1

    node_bufs = 6 if not first else 0
    out_types = [jax.ShapeDtypeStruct((NPAD,), jnp.float32)] * 2
    if not first:
        out_types.append(jax.ShapeDtypeStruct((NPAD,), jnp.float32))
    scratch = [
        pltpu.VMEM((CHUNK,), jnp.int32),
        pltpu.VMEM((CHUNK,), jnp.int32),
        pltpu.VMEM((CHUNK,), jnp.float32),
        pltpu.VMEM((CHUNK,), jnp.float32),
        pltpu.VMEM((NSLICE,), jnp.float32),
    ]
    scratch += [pltpu.VMEM((NSLICE,), jnp.float32)] * node_bufs
    if not first:
        scratch.append(pltpu.VMEM((L,), jnp.float32))
    scratch += [
        pltpu.VMEM_SHARED((NPAD,), jnp.float32),
        pltpu.VMEM_SHARED((NPAD,), jnp.float32),
        pltpu.SemaphoreType.DMA,
    ]
    return pl.kernel(
        body,
        out_type=out_types,
        mesh=_mesh,
        scratch_types=scratch,
    )


_prop_first = _make_prop(True)
_prop_rest = _make_prop(False)


_EW_RROWS = E // 128
_EW_BLOCK = 200
_EW_EDGES = _EW_BLOCK * 128
_EW_GRID = _EW_RROWS // _EW_BLOCK


def _ew_body(a_ref, sw_ref, b_ref, ei_ref, o_ref, row_ref, col_ref):
    acc = jnp.dot(a_ref[...], sw_ref[...],
                  preferred_element_type=jnp.float32,
                  precision=lax.Precision.HIGHEST)
    o_ref[...] = jax.nn.sigmoid(acc + b_ref[0])
    row_ref[...] = ei_ref[0, :]
    col_ref[...] = ei_ref[1, :]


_ew_call = pl.pallas_call(
    _ew_body,
    grid=(_EW_GRID,),
    in_specs=[
        pl.BlockSpec((_EW_BLOCK, 2048), lambda i: (i, 0)),
        pl.BlockSpec((2048, 128), lambda i: (0, 0)),
        pl.BlockSpec(memory_space=pltpu.SMEM),
        pl.BlockSpec((2, _EW_EDGES), lambda i: (0, i)),
    ],
    out_specs=[
        pl.BlockSpec((_EW_BLOCK, 128), lambda i: (i, 0)),
        pl.BlockSpec((_EW_EDGES,), lambda i: (i,)),
        pl.BlockSpec((_EW_EDGES,), lambda i: (i,)),
    ],
    out_shape=[
        jax.ShapeDtypeStruct((_EW_RROWS, 128), jnp.float32),
        jax.ShapeDtypeStruct((E,), jnp.int32),
        jax.ShapeDtypeStruct((E,), jnp.int32),
    ],
    compiler_params=pltpu.CompilerParams(
        dimension_semantics=("parallel",)),
)


def _prep_body(p0_ref, p1_ref, m_ref, dis_ref, dinv_ref, h_ref, y_ref):
    deg = p0_ref[...] + p1_ref[...] + 1.0
    dinv = 1.0 / deg
    dis = lax.rsqrt(deg)
    h = jnp.maximum(m_ref[...], 0.0)
    dis_ref[...] = dis
    dinv_ref[...] = dinv
    h_ref[...] = h
    y_ref[...] = dis * h


_prep_call = pl.pallas_call(
    _prep_body,
    out_shape=[jax.ShapeDtypeStruct((NROWS, 128), jnp.float32)] * 4,
)


def _finish_body(alpha_ref, bias_ref, a0_ref, a1_ref, x_ref, dis_ref,
                 dinv_ref, h_ref, o_ref):
    alpha = alpha_ref[0]
    agg = dis_ref[...] * (a0_ref[...] + a1_ref[...]) + x_ref[...] * dinv_ref[...]
    xn = agg * (1.0 - alpha) + alpha * h_ref[...]
    o_ref[...] = jnp.tanh(xn - jax.nn.softplus(bias_ref[0]))


_finish_call = pl.pallas_call(
    _finish_body,
    in_specs=[
        pl.BlockSpec(memory_space=pltpu.SMEM),
        pl.BlockSpec(memory_space=pltpu.SMEM),
    ] + [pl.BlockSpec((NROWS, 128), lambda: (0, 0))] * 6,
    out_shape=jax.ShapeDtypeStruct((NROWS, 128), jnp.float32),
)


def kernel(edge_attr, mask, edge_index, W_edge, b_edge, alpha, bias):
    ei = edge_index.astype(jnp.int32)
    b1 = b_edge.astype(jnp.float32).reshape(1)
    alpha1 = alpha.astype(jnp.float32).reshape(1)
    bias1 = bias.astype(jnp.float32).reshape(1)

    a2d = edge_attr.reshape(_EW_RROWS, 2048)
    sw = jnp.kron(jnp.eye(128, dtype=jnp.float32), W_edge.astype(jnp.float32))
    ew2d, row, col = _ew_call(a2d, sw, b1, ei)
    ew = ew2d.reshape(E)

    d0, d1 = _deg_call(col, ew)
    maskp = jnp.pad(mask.reshape(N), (0, NPAD - N)).reshape(NROWS, 128)
    dis, dinv, h, y = _prep_call(d0.reshape(NROWS, 128),
                                 d1.reshape(NROWS, 128), maskp)

    disf = dis.reshape(NPAD)
    dinvf = dinv.reshape(NPAD)
    hf = h.reshape(NPAD)
    a0, a1 = _prop_first(y.reshape(NPAD), row, col, ew)
    x = hf
    alpha16 = jnp.broadcast_to(alpha1, (L,))
    for _ in range(K - 1):
        a0, a1, x = _prop_rest(alpha16, a0, a1, x, disf, dinvf, hf,
                               row, col, ew)

    out = _finish_call(alpha1, bias1, a0.reshape(NROWS, 128),
                       a1.reshape(NROWS, 128), x.reshape(NROWS, 128),
                       dis, dinv, h)
    fill = out.reshape(NPAD)[:N].reshape(N, 1)
    return (fill, ew)


# speedup vs baseline: 1.6705x; 1.6705x over previous
import jax
import jax.numpy as jnp
from jax import lax
from jax.experimental import pallas as pl
from jax.experimental.pallas import tpu as pltpu
from jax.experimental.pallas import tpu_sc as plsc

N = 100000
E = 3200000
EDGE_DIM = 16
K = 5

NC, NS, L = 2, 16, 16
NW = NC * NS
NPAD = 100352
E_PER_TILE = E // NW
CHUNK = 10000
N_CHUNKS = E_PER_TILE // CHUNK
NSLICE = NPAD // NS
NROWS = NPAD // 128

_mesh = plsc.VectorSubcoreMesh(core_axis_name="c", subcore_axis_name="s")


def _fill_zero(zv):
    zero = jnp.zeros((L,), jnp.float32)

    def zloop(i, carry):
        zv[pl.ds(i * L, L)] = zero
        return carry

    lax.fori_loop(0, NSLICE // L, zloop, 0)


def _scatter_chunks(wid, row_hbm, col_hbm, w_hbm, row_v, col_v, w_v,
                    gath_v, y_sh, acc, sem, gather):
    base = wid * E_PER_TILE

    def chunk(j, carry):
        e0 = base + j * CHUNK
        if gather:
            c1 = pltpu.async_copy(row_hbm.at[pl.ds(e0, CHUNK)], row_v, sem)
        c3 = pltpu.async_copy(w_hbm.at[pl.ds(e0, CHUNK)], w_v, sem)
        c2 = pltpu.async_copy(col_hbm.at[pl.ds(e0, CHUNK)], col_v, sem)
        if gather:
            c1.wait()
            pltpu.sync_copy(y_sh.at[row_v], gath_v)
            c3.wait()

            def mul_group(g, c_):
                s = pl.ds(g * L, L)
                gath_v[s] = gath_v[s] * w_v[s]
                return c_

            lax.fori_loop(0, CHUNK // L, mul_group, 0)
            src = gath_v
        else:
            c3.wait()
            src = w_v
        c2.wait()
        pltpu.sync_copy(src, acc.at[col_v], add=True)
        return carry

    lax.fori_loop(0, N_CHUNKS, chunk, 0)


def _writeback(cid, sid, acc, out0, out1):
    sl = pl.ds(sid * NSLICE, NSLICE)

    @pl.when(cid == 0)
    def _():
        pltpu.sync_copy(acc.at[sl], out0.at[sl])

    @pl.when(cid == 1)
    def _():
        pltpu.sync_copy(acc.at[sl], out1.at[sl])


def _deg_body(col_hbm, w_hbm, out0, out1, col_v, w_v, zv, acc, sem):
    cid = lax.axis_index("c")
    sid = lax.axis_index("s")
    wid = cid * NS + sid
    _fill_zero(zv)
    pltpu.sync_copy(zv, acc.at[pl.ds(sid * NSLICE, NSLICE)])
    plsc.subcore_barrier()
    _scatter_chunks(wid, None, col_hbm, w_hbm, None, col_v, w_v,
                    None, None, acc, sem, gather=False)
    plsc.subcore_barrier()
    _writeback(cid, sid, acc, out0, out1)


_deg_call = pl.kernel(
    _deg_body,
    out_type=[jax.ShapeDtypeStruct((NPAD,), jnp.float32)] * 2,
    mesh=_mesh,
    scratch_types=[
        pltpu.VMEM((CHUNK,), jnp.int32),
        pltpu.VMEM((CHUNK,), jnp.float32),
        pltpu.VMEM((NSLICE,), jnp.float32),
        pltpu.VMEM_SHARED((NPAD,), jnp.float32),
        pltpu.SemaphoreType.DMA,
    ],
)


def _make_prop(first):
    if first:
        def body(y_hbm, row_hbm, col_hbm, w_hbm, out0, out1,
                 row_v, col_v, w_v, gath_v, zv, y_sh, acc, sem):
            cid = lax.axis_index("c")
            sid = lax.axis_index("s")
            wid = cid * NS + sid
            sl = pl.ds(sid * NSLICE, NSLICE)
            pltpu.sync_copy(y_hbm.at[sl], zv)
            pltpu.sync_copy(zv, y_sh.at[sl])
            _fill_zero(zv)
            pltpu.sync_copy(zv, acc.at[sl])
            plsc.subcore_barrier()
            _scatter_chunks(wid, row_hbm, col_hbm, w_hbm, row_v, col_v,
                            w_v, gath_v, y_sh, acc, sem, gather=True)
            plsc.subcore_barrier()
            _writeback(cid, sid, acc, out0, out1)

        extra_in = 0
    else:
        def body(alpha_hbm, a0_hbm, a1_hbm, xp_hbm, dis_hbm, dinv_hbm,
                 h_hbm, row_hbm, col_hbm, w_hbm, out0, out1, xn_hbm,
                 row_v, col_v, w_v, gath_v, zv,
                 a0_v, a1_v, xp_v, dis_v, dinv_v, h_v, av,
                 y_sh, acc, sem):
            cid = lax.axis_index("c")
            sid = lax.axis_index("s")
            wid = cid * NS + sid
            sl = pl.ds(sid * NSLICE, NSLICE)
            pltpu.sync_copy(alpha_hbm, av)
            d1 = pltpu.async_copy(a0_hbm.at[sl], a0_v, sem)
            d2 = pltpu.async_copy(a1_hbm.at[sl], a1_v, sem)
            d3 = pltpu.async_copy(xp_hbm.at[sl], xp_v, sem)
            d4 = pltpu.async_copy(dis_hbm.at[sl], dis_v, sem)
            d5 = pltpu.async_copy(dinv_hbm.at[sl], dinv_v, sem)
            d6 = pltpu.async_copy(h_hbm.at[sl], h_v, sem)
            alpha = av[pl.ds(0, L)]
            one_m_alpha = 1.0 - alpha
            d1.wait(); d2.wait(); d3.wait(); d4.wait(); d5.wait(); d6.wait()

            def up_group(g, carry):
                s = pl.ds(g * L, L)
                agg = dis_v[s] * (a0_v[s] + a1_v[s]) + xp_v[s] * dinv_v[s]
                xn = agg * one_m_alpha + alpha * h_v[s]
                xp_v[s] = xn
                a0_v[s] = dis_v[s] * xn
                return carry

            lax.fori_loop(0, NSLICE // L, up_group, 0)
            pltpu.sync_copy(a0_v, y_sh.at[sl])

            @pl.when(cid == 0)
            def _():
                pltpu.sync_copy(xp_v, xn_hbm.at[sl])

            _fill_zero(zv)
            pltpu.sync_copy(zv, acc.at[sl])
            plsc.subcore_barrier()
            _scatter_chunks(wid, row_hbm, col_hbm, w_hbm, row_v, col_v,
                            w_v, gath_v, y_sh, acc, sem, gather=True)
            plsc.subcore_barrier()
            _writeback(cid, sid, acc, out0, out1)

        extra_in = 1

    node_bufs = 6 if not first else 0
    out_types = [jax.ShapeDtypeStruct((NPAD,), jnp.float32)] * 2
    if not first:
        out_types.append(jax.ShapeDtypeStruct((NPAD,), jnp.float32))
    scratch = [
        pltpu.VMEM((CHUNK,), jnp.int32),
        pltpu.VMEM((CHUNK,), jnp.int32),
        pltpu.VMEM((CHUNK,), jnp.float32),
        pltpu.VMEM((CHUNK,), jnp.float32),
        pltpu.VMEM((NSLICE,), jnp.float32),
    ]
    scratch += [pltpu.VMEM((NSLICE,), jnp.float32)] * node_bufs
    if not first:
        scratch.append(pltpu.VMEM((L,), jnp.float32))
    scratch += [
        pltpu.VMEM_SHARED((NPAD,), jnp.float32),
        pltpu.VMEM_SHARED((NPAD,), jnp.float32),
        pltpu.SemaphoreType.DMA,
    ]
    return pl.kernel(
        body,
        out_type=out_types,
        mesh=_mesh,
        scratch_types=scratch,
    )


_prop_first = _make_prop(True)
_prop_rest = _make_prop(False)


_EW_RROWS = E // 128
_EW_BLOCK = 200
_EW_GRID = _EW_RROWS // _EW_BLOCK


def _ew_body(a_ref, w_ref, b_ref, o_ref):
    s = jnp.sum(a_ref[...] * w_ref[...], axis=2) + b_ref[0]
    o_ref[...] = jax.nn.sigmoid(s)


_ew_call = pl.pallas_call(
    _ew_body,
    grid=(_EW_GRID,),
    in_specs=[
        pl.BlockSpec((_EW_BLOCK, 128, EDGE_DIM), lambda i: (i, 0, 0)),
        pl.BlockSpec((1, 1, EDGE_DIM), lambda i: (0, 0, 0)),
        pl.BlockSpec(memory_space=pltpu.SMEM),
    ],
    out_specs=pl.BlockSpec((_EW_BLOCK, 128), lambda i: (i, 0)),
    out_shape=jax.ShapeDtypeStruct((_EW_RROWS, 128), jnp.float32),
    compiler_params=pltpu.CompilerParams(
        dimension_semantics=("parallel",)),
)


_IX_BLOCK = 25600
_IX_GRID = E // _IX_BLOCK


def _ix_body(ei_ref, row_ref, col_ref):
    row_ref[...] = ei_ref[0, :]
    col_ref[...] = ei_ref[1, :]


_ix_call = pl.pallas_call(
    _ix_body,
    grid=(_IX_GRID,),
    in_specs=[pl.BlockSpec((2, _IX_BLOCK), lambda i: (0, i))],
    out_specs=[
        pl.BlockSpec((_IX_BLOCK,), lambda i: (i,)),
        pl.BlockSpec((_IX_BLOCK,), lambda i: (i,)),
    ],
    out_shape=[
        jax.ShapeDtypeStruct((E,), jnp.int32),
        jax.ShapeDtypeStruct((E,), jnp.int32),
    ],
    compiler_params=pltpu.CompilerParams(
        dimension_semantics=("parallel",)),
)


def _prep_body(p0_ref, p1_ref, m_ref, dis_ref, dinv_ref, h_ref, y_ref):
    deg = p0_ref[...] + p1_ref[...] + 1.0
    dinv = 1.0 / deg
    dis = lax.rsqrt(deg)
    h = jnp.maximum(m_ref[...], 0.0)
    dis_ref[...] = dis
    dinv_ref[...] = dinv
    h_ref[...] = h
    y_ref[...] = dis * h


_prep_call = pl.pallas_call(
    _prep_body,
    out_shape=[jax.ShapeDtypeStruct((NROWS, 128), jnp.float32)] * 4,
)


def _finish_body(alpha_ref, bias_ref, a0_ref, a1_ref, x_ref, dis_ref,
                 dinv_ref, h_ref, o_ref):
    alpha = alpha_ref[0]
    agg = dis_ref[...] * (a0_ref[...] + a1_ref[...]) + x_ref[...] * dinv_ref[...]
    xn = agg * (1.0 - alpha) + alpha * h_ref[...]
    o_ref[...] = jnp.tanh(xn - jax.nn.softplus(bias_ref[0]))


_finish_call = pl.pallas_call(
    _finish_body,
    in_specs=[
        pl.BlockSpec(memory_space=pltpu.SMEM),
        pl.BlockSpec(memory_space=pltpu.SMEM),
    ] + [pl.BlockSpec((NROWS, 128), lambda: (0, 0))] * 6,
    out_shape=jax.ShapeDtypeStruct((NROWS, 128), jnp.float32),
)


def kernel(edge_attr, mask, edge_index, W_edge, b_edge, alpha, bias):
    ei = edge_index.astype(jnp.int32)
    b1 = b_edge.astype(jnp.float32).reshape(1)
    alpha1 = alpha.astype(jnp.float32).reshape(1)
    bias1 = bias.astype(jnp.float32).reshape(1)

    a3 = edge_attr.reshape(_EW_RROWS, 128, EDGE_DIM)
    w3 = W_edge.astype(jnp.float32).reshape(1, 1, EDGE_DIM)
    ew = _ew_call(a3, w3, b1).reshape(E)
    row, col = _ix_call(ei)

    d0, d1 = _deg_call(col, ew)
    maskp = jnp.pad(mask.reshape(N), (0, NPAD - N)).reshape(NROWS, 128)
    dis, dinv, h, y = _prep_call(d0.reshape(NROWS, 128),
                                 d1.reshape(NROWS, 128), maskp)

    disf = dis.reshape(NPAD)
    dinvf = dinv.reshape(NPAD)
    hf = h.reshape(NPAD)
    a0, a1 = _prop_first(y.reshape(NPAD), row, col, ew)
    x = hf
    alpha16 = jnp.broadcast_to(alpha1, (L,))
    for _ in range(K - 1):
        a0, a1, x = _prop_rest(alpha16, a0, a1, x, disf, dinvf, hf,
                               row, col, ew)

    out = _finish_call(alpha1, bias1, a0.reshape(NROWS, 128),
                       a1.reshape(NROWS, 128), x.reshape(NROWS, 128),
                       dis, dinv, h)
    fill = out.reshape(NPAD)[:N].reshape(N, 1)
    return (fill, ew)


# speedup vs baseline: 2.7311x; 1.6350x over previous
import jax
import jax.numpy as jnp
from jax import lax
from jax.experimental import pallas as pl
from jax.experimental.pallas import tpu as pltpu
from jax.experimental.pallas import tpu_sc as plsc

N = 100000
E = 3200000
EDGE_DIM = 16
K = 5

NC, NS, L = 2, 16, 16
NW = NC * NS
NPAD = 100352
E_PER_TILE = E // NW
CHUNK = 10000
N_CHUNKS = E_PER_TILE // CHUNK
NSLICE = NPAD // NS
NROWS = NPAD // 128

_mesh = plsc.VectorSubcoreMesh(core_axis_name="c", subcore_axis_name="s")


def _fill_zero(zv):
    zero = jnp.zeros((L,), jnp.float32)

    def zloop(i, carry):
        zv[pl.ds(i * L, L)] = zero
        return carry

    lax.fori_loop(0, NSLICE // L, zloop, 0)


def _scatter_chunks(wid, row_hbm, col_hbm, w_hbm, row_v, col_v, w_v,
                    gath_v, y_sh, acc, sem, gather):
    base = wid * E_PER_TILE

    def chunk(j, carry):
        e0 = base + j * CHUNK
        if gather:
            c1 = pltpu.async_copy(row_hbm.at[pl.ds(e0, CHUNK)], row_v, sem)
        c3 = pltpu.async_copy(w_hbm.at[pl.ds(e0, CHUNK)], w_v, sem)
        c2 = pltpu.async_copy(col_hbm.at[pl.ds(e0, CHUNK)], col_v, sem)
        if gather:
            c1.wait()
            pltpu.sync_copy(y_sh.at[row_v], gath_v)
            c3.wait()

            def mul_group(g, c_):
                s = pl.ds(g * L, L)
                gath_v[s] = gath_v[s] * w_v[s]
                return c_

            lax.fori_loop(0, CHUNK // L, mul_group, 0)
            src = gath_v
        else:
            c3.wait()
            src = w_v
        c2.wait()
        pltpu.sync_copy(src, acc.at[col_v], add=True)
        return carry

    lax.fori_loop(0, N_CHUNKS, chunk, 0)


def _writeback(cid, sid, acc, out0, out1):
    sl = pl.ds(sid * NSLICE, NSLICE)

    @pl.when(cid == 0)
    def _():
        pltpu.sync_copy(acc.at[sl], out0.at[sl])

    @pl.when(cid == 1)
    def _():
        pltpu.sync_copy(acc.at[sl], out1.at[sl])


def _deg_body(col_hbm, w_hbm, out0, out1, col_v, w_v, zv, acc, sem):
    cid = lax.axis_index("c")
    sid = lax.axis_index("s")
    wid = cid * NS + sid
    _fill_zero(zv)
    pltpu.sync_copy(zv, acc.at[pl.ds(sid * NSLICE, NSLICE)])
    plsc.subcore_barrier()
    _scatter_chunks(wid, None, col_hbm, w_hbm, None, col_v, w_v,
                    None, None, acc, sem, gather=False)
    plsc.subcore_barrier()
    _writeback(cid, sid, acc, out0, out1)


_deg_call = pl.kernel(
    _deg_body,
    out_type=[jax.ShapeDtypeStruct((NPAD,), jnp.float32)] * 2,
    mesh=_mesh,
    scratch_types=[
        pltpu.VMEM((CHUNK,), jnp.int32),
        pltpu.VMEM((CHUNK,), jnp.float32),
        pltpu.VMEM((NSLICE,), jnp.float32),
        pltpu.VMEM_SHARED((NPAD,), jnp.float32),
        pltpu.SemaphoreType.DMA,
    ],
)


def _make_prop(first):
    if first:
        def body(y_hbm, row_hbm, col_hbm, w_hbm, out0, out1,
                 row_v, col_v, w_v, gath_v, zv, y_sh, acc, sem):
            cid = lax.axis_index("c")
            sid = lax.axis_index("s")
            wid = cid * NS + sid
            sl = pl.ds(sid * NSLICE, NSLICE)
            pltpu.sync_copy(y_hbm.at[sl], zv)
            pltpu.sync_copy(zv, y_sh.at[sl])
            _fill_zero(zv)
            pltpu.sync_copy(zv, acc.at[sl])
            plsc.subcore_barrier()
            _scatter_chunks(wid, row_hbm, col_hbm, w_hbm, row_v, col_v,
                            w_v, gath_v, y_sh, acc, sem, gather=True)
            plsc.subcore_barrier()
            _writeback(cid, sid, acc, out0, out1)

        extra_in = 0
    else:
        def body(alpha_hbm, a0_hbm, a1_hbm, xp_hbm, dis_hbm, dinv_hbm,
                 h_hbm, row_hbm, col_hbm, w_hbm, out0, out1, xn_hbm,
                 row_v, col_v, w_v, gath_v, zv,
                 a0_v, a1_v, xp_v, dis_v, dinv_v, h_v, av,
                 y_sh, acc, sem):
            cid = lax.axis_index("c")
            sid = lax.axis_index("s")
            wid = cid * NS + sid
            sl = pl.ds(sid * NSLICE, NSLICE)
            pltpu.sync_copy(alpha_hbm, av)
            d1 = pltpu.async_copy(a0_hbm.at[sl], a0_v, sem)
            d2 = pltpu.async_copy(a1_hbm.at[sl], a1_v, sem)
            d3 = pltpu.async_copy(xp_hbm.at[sl], xp_v, sem)
            d4 = pltpu.async_copy(dis_hbm.at[sl], dis_v, sem)
            d5 = pltpu.async_copy(dinv_hbm.at[sl], dinv_v, sem)
            d6 = pltpu.async_copy(h_hbm.at[sl], h_v, sem)
            alpha = av[pl.ds(0, L)]
            one_m_alpha = 1.0 - alpha
            d1.wait(); d2.wait(); d3.wait(); d4.wait(); d5.wait(); d6.wait()

            def up_group(g, carry):
                s = pl.ds(g * L, L)
                agg = dis_v[s] * (a0_v[s] + a1_v[s]) + xp_v[s] * dinv_v[s]
                xn = agg * one_m_alpha + alpha * h_v[s]
                xp_v[s] = xn
                a0_v[s] = dis_v[s] * xn
                return carry

            lax.fori_loop(0, NSLICE // L, up_group, 0)
            pltpu.sync_copy(a0_v, y_sh.at[sl])

            @pl.when(cid == 0)
            def _():
                pltpu.sync_copy(xp_v, xn_hbm.at[sl])

            _fill_zero(zv)
            pltpu.sync_copy(zv, acc.at[sl])
            plsc.subcore_barrier()
            _scatter_chunks(wid, row_hbm, col_hbm, w_hbm, row_v, col_v,
                            w_v, gath_v, y_sh, acc, sem, gather=True)
            plsc.subcore_barrier()
            _writeback(cid, sid, acc, out0, out1)

        extra_in = 1

    node_bufs = 6 if not first else 0
    out_types = [jax.ShapeDtypeStruct((NPAD,), jnp.float32)] * 2
    if not first:
        out_types.append(jax.ShapeDtypeStruct((NPAD,), jnp.float32))
    scratch = [
        pltpu.VMEM((CHUNK,), jnp.int32),
        pltpu.VMEM((CHUNK,), jnp.int32),
        pltpu.VMEM((CHUNK,), jnp.float32),
        pltpu.VMEM((CHUNK,), jnp.float32),
        pltpu.VMEM((NSLICE,), jnp.float32),
    ]
    scratch += [pltpu.VMEM((NSLICE,), jnp.float32)] * node_bufs
    if not first:
        scratch.append(pltpu.VMEM((L,), jnp.float32))
    scratch += [
        pltpu.VMEM_SHARED((NPAD,), jnp.float32),
        pltpu.VMEM_SHARED((NPAD,), jnp.float32),
        pltpu.SemaphoreType.DMA,
    ]
    return pl.kernel(
        body,
        out_type=out_types,
        mesh=_mesh,
        scratch_types=scratch,
    )


_prop_first = _make_prop(True)
_prop_rest = _make_prop(False)


_EW_BLK = 25600
_EW_GRID = E // _EW_BLK


def _ew_body(a_ref, w_ref, b_ref, o_ref):
    s = jnp.sum(a_ref[...] * w_ref[...], axis=0) + b_ref[0]
    o_ref[...] = 0.5 * jnp.tanh(0.5 * s) + 0.5


_ew_call = pl.pallas_call(
    _ew_body,
    grid=(_EW_GRID,),
    in_specs=[
        pl.BlockSpec((EDGE_DIM, _EW_BLK), lambda i: (0, i)),
        pl.BlockSpec((EDGE_DIM, 1), lambda i: (0, 0)),
        pl.BlockSpec(memory_space=pltpu.SMEM),
    ],
    out_specs=pl.BlockSpec((_EW_BLK,), lambda i: (i,)),
    out_shape=jax.ShapeDtypeStruct((E,), jnp.float32),
    compiler_params=pltpu.CompilerParams(
        dimension_semantics=("parallel",)),
)


_IX_BLOCK = 25600
_IX_GRID = E // _IX_BLOCK


def _ix_body(ei_ref, row_ref, col_ref):
    row_ref[...] = ei_ref[0, :]
    col_ref[...] = ei_ref[1, :]


_ix_call = pl.pallas_call(
    _ix_body,
    grid=(_IX_GRID,),
    in_specs=[pl.BlockSpec((2, _IX_BLOCK), lambda i: (0, i))],
    out_specs=[
        pl.BlockSpec((_IX_BLOCK,), lambda i: (i,)),
        pl.BlockSpec((_IX_BLOCK,), lambda i: (i,)),
    ],
    out_shape=[
        jax.ShapeDtypeStruct((E,), jnp.int32),
        jax.ShapeDtypeStruct((E,), jnp.int32),
    ],
    compiler_params=pltpu.CompilerParams(
        dimension_semantics=("parallel",)),
)


def _prep_body(p0_ref, p1_ref, m_ref, dis_ref, dinv_ref, h_ref, y_ref):
    deg = p0_ref[...] + p1_ref[...] + 1.0
    dinv = 1.0 / deg
    dis = lax.rsqrt(deg)
    h = jnp.maximum(m_ref[...], 0.0)
    dis_ref[...] = dis
    dinv_ref[...] = dinv
    h_ref[...] = h
    y_ref[...] = dis * h


_prep_call = pl.pallas_call(
    _prep_body,
    out_shape=[jax.ShapeDtypeStruct((NROWS, 128), jnp.float32)] * 4,
)


def _finish_body(alpha_ref, bias_ref, a0_ref, a1_ref, x_ref, dis_ref,
                 dinv_ref, h_ref, o_ref):
    alpha = alpha_ref[0]
    agg = dis_ref[...] * (a0_ref[...] + a1_ref[...]) + x_ref[...] * dinv_ref[...]
    xn = agg * (1.0 - alpha) + alpha * h_ref[...]
    o_ref[...] = jnp.tanh(xn - jax.nn.softplus(bias_ref[0]))


_finish_call = pl.pallas_call(
    _finish_body,
    in_specs=[
        pl.BlockSpec(memory_space=pltpu.SMEM),
        pl.BlockSpec(memory_space=pltpu.SMEM),
    ] + [pl.BlockSpec((NROWS, 128), lambda: (0, 0))] * 6,
    out_shape=jax.ShapeDtypeStruct((NROWS, 128), jnp.float32),
)


def kernel(edge_attr, mask, edge_index, W_edge, b_edge, alpha, bias):
    ei = edge_index.astype(jnp.int32)
    b1 = b_edge.astype(jnp.float32).reshape(1)
    alpha1 = alpha.astype(jnp.float32).reshape(1)
    bias1 = bias.astype(jnp.float32).reshape(1)

    ew = _ew_call(edge_attr.T, W_edge.astype(jnp.float32), b1)
    row, col = _ix_call(ei)

    d0, d1 = _deg_call(col, ew)
    maskp = jnp.pad(mask.reshape(N), (0, NPAD - N)).reshape(NROWS, 128)
    dis, dinv, h, y = _prep_call(d0.reshape(NROWS, 128),
                                 d1.reshape(NROWS, 128), maskp)

    disf = dis.reshape(NPAD)
    dinvf = dinv.reshape(NPAD)
    hf = h.reshape(NPAD)
    a0, a1 = _prop_first(y.reshape(NPAD), row, col, ew)
    x = hf
    alpha16 = jnp.broadcast_to(alpha1, (L,))
    for _ in range(K - 1):
        a0, a1, x = _prop_rest(alpha16, a0, a1, x, disf, dinvf, hf,
                               row, col, ew)

    out = _finish_call(alpha1, bias1, a0.reshape(NROWS, 128),
                       a1.reshape(NROWS, 128), x.reshape(NROWS, 128),
                       dis, dinv, h)
    fill = out.reshape(NPAD)[:N].reshape(N, 1)
    return (fill, ew)
